# Initial kernel scaffold; baseline (speedup 1.0000x reference)
import jax
import jax.numpy as jnp
from jax.experimental import pallas as pl


def kernel(context, center, negatives, ctx_table, ctr_table):
    raise NotImplementedError("write your pallas kernel here")


# speedup vs baseline: 3.3631x; 3.3631x over previous
import functools

import jax
import jax.numpy as jnp
from jax import lax
from jax.experimental import pallas as pl
from jax.experimental.pallas import tpu as pltpu
from jax.experimental.pallas import tpu_sc as plsc

_B = 16384
_D = 64
_CTX = 20
_NEG = 20
_NC = 2
_NS = 16
_NW = _NC * _NS
_PER_W = _B // _NW
_CH = 32
_NCHUNK = _PER_W // _CH
_ROWS = _CH * _CTX
_IDXW = 128
_KD = _ROWS // _IDXW
_GPC = _CH // 16


def _sc_scores(ctx_idx2d, center, neg_idx2d, ctx_table, ctr_table):
  mesh = plsc.VectorSubcoreMesh(core_axis_name="c", subcore_axis_name="s")

  @functools.partial(
      pl.kernel,
      out_type=(
          jax.ShapeDtypeStruct((_NW, _PER_W), jnp.float32),
          jax.ShapeDtypeStruct((_NW, _NEG, _PER_W), jnp.float32),
      ),
      mesh=mesh,
      compiler_params=pltpu.CompilerParams(
          needs_layout_passes=False, use_tc_tiling_on_sc=False),
      scratch_types=[
          pltpu.VMEM((_ROWS,), jnp.int32),
          pltpu.VMEM((_ROWS,), jnp.int32),
          pltpu.VMEM((_CH,), jnp.int32),
          pltpu.VMEM((_ROWS, _D), jnp.float32),
          pltpu.VMEM((_ROWS, _D), jnp.float32),
          pltpu.VMEM((_CH, _D), jnp.float32),
          pltpu.VMEM((_PER_W,), jnp.float32),
          pltpu.VMEM((_NEG, _PER_W), jnp.float32),
          pltpu.SemaphoreType.DMA,
      ],
  )
  def scores(ctx_i_hbm, ctr_i_hbm, neg_i_hbm, ctx_t_hbm, ctr_t_hbm,
             pos_hbm, negs_hbm,
             ctx_idx_v, neg_idx_v, ctr_idx_v,
             ctx_rows_v, neg_rows_v, ctr_rows_v,
             pos_v, negs_v, sem):
    wid = lax.axis_index("s") * _NC + lax.axis_index("c")
    lane = lax.iota(jnp.int32, 16)

    def chunk_body(c, carry):
      ebase = wid * _PER_W + c * _CH
      ibase = ebase * _CTX
      pltpu.sync_copy(ctx_i_hbm.at[pl.ds(ibase, _ROWS)], ctx_idx_v)
      pltpu.sync_copy(neg_i_hbm.at[pl.ds(ibase, _ROWS)], neg_idx_v)
      pltpu.sync_copy(ctr_i_hbm.at[pl.ds(ebase, _CH)], ctr_idx_v)
      copies = []
      for k in range(_KD):
        copies.append(pltpu.async_copy(
            ctx_t_hbm.at[ctx_idx_v.at[pl.ds(k * _IDXW, _IDXW)]],
            ctx_rows_v.at[pl.ds(k * _IDXW, _IDXW)], sem))
        copies.append(pltpu.async_copy(
            ctr_t_hbm.at[neg_idx_v.at[pl.ds(k * _IDXW, _IDXW)]],
            neg_rows_v.at[pl.ds(k * _IDXW, _IDXW)], sem))
      copies.append(pltpu.async_copy(ctr_t_hbm.at[ctr_idx_v], ctr_rows_v, sem))
      for cp in copies:
        cp.wait()

      for g in range(_GPC):
        rows_base = g * (16 * _CTX) + lane * _CTX
        rows_cn = [rows_base + n for n in range(_CTX)]
        rows_ctr = g * 16 + lane

        def j_body(j, acc, rows_cn=rows_cn, rows_ctr=rows_ctr):
          pos_acc, neg_accs = acc
          col = jnp.full((16,), j, jnp.int32)
          s = plsc.load_gather(ctx_rows_v, [rows_cn[0], col])
          for cc in range(1, _CTX):
            s = s + plsc.load_gather(ctx_rows_v, [rows_cn[cc], col])
          m = s * jnp.float32(1.0 / _CTX)
          cv = plsc.load_gather(ctr_rows_v, [rows_ctr, col])
          pos_acc = pos_acc + m * cv
          neg_accs = tuple(
              neg_accs[n] + m * plsc.load_gather(neg_rows_v, [rows_cn[n], col])
              for n in range(_NEG))
          return pos_acc, neg_accs

        zero = jnp.zeros((16,), jnp.float32)
        pos_acc, neg_accs = lax.fori_loop(
            0, _D, j_body, (zero, tuple(zero for _ in range(_NEG))))
        off = c * _CH + g * 16
        pos_v[pl.ds(off, 16)] = pos_acc
        for n in range(_NEG):
          negs_v[n, pl.ds(off, 16)] = neg_accs[n]
      return carry

    lax.fori_loop(0, _NCHUNK, chunk_body, jnp.int32(0))
    pltpu.sync_copy(pos_v, pos_hbm.at[wid])
    pltpu.sync_copy(negs_v, negs_hbm.at[wid])

  return scores(ctx_idx2d, center, neg_idx2d, ctx_table, ctr_table)


def _loss_tc(pos, negs):
  def body(pos_ref, neg_ref, out_ref):
    p = pos_ref[...]
    q = neg_ref[...]

    def ls(x):
      return jnp.minimum(x, 0.0) - jnp.log1p(jnp.exp(-jnp.abs(x)))

    total = jnp.sum(ls(p)) + jnp.sum(ls(-q))
    out_ref[...] = jnp.full((1, 1), -total / _B, jnp.float32)

  return pl.pallas_call(
      body,
      out_shape=jax.ShapeDtypeStruct((1, 1), jnp.float32),
  )(pos, negs)


def kernel(context, center, negatives, ctx_table, ctr_table):
  ctx_i = context.astype(jnp.int32).reshape(_B * _CTX)
  neg_i = negatives.astype(jnp.int32).reshape(_B * _NEG)
  ctr_i = center.astype(jnp.int32)
  pos, negs = _sc_scores(ctx_i, ctr_i, neg_i, ctx_table, ctr_table)
  loss = _loss_tc(pos, negs.reshape(_NW * _NEG, _PER_W))
  return loss[0, 0]


# speedup vs baseline: 5.9292x; 1.7630x over previous
import functools

import jax
import jax.numpy as jnp
from jax import lax
from jax.experimental import pallas as pl
from jax.experimental.pallas import tpu as pltpu
from jax.experimental.pallas import tpu_sc as plsc

_V = 1000000
_B = 16384
_D = 64
_CTX = 20
_NEG = 20
_NC = 2
_NS = 16
_NW = _NC * _NS
_PER_W = _B // _NW
_CH = 16
_NCHUNK = _PER_W // _CH
_ROWS = _CH * _CTX
_IPW = _PER_W * _CTX
_QS = _D // 16
_SPLITS = ((0, 128), (128, 128), (256, 64))


def _sc_scores(ctx_idx, center_idx, neg_idx, ctx_t2, ctr_t2):
  mesh = plsc.VectorSubcoreMesh(core_axis_name="c", subcore_axis_name="s")

  @functools.partial(
      pl.kernel,
      out_type=(
          jax.ShapeDtypeStruct((_NW, _PER_W), jnp.float32),
          jax.ShapeDtypeStruct((_NW, _NEG, _PER_W), jnp.float32),
      ),
      mesh=mesh,
      compiler_params=pltpu.CompilerParams(
          needs_layout_passes=False, use_tc_tiling_on_sc=False),
      scratch_types=[
          pltpu.VMEM((_IPW,), jnp.int32),
          pltpu.VMEM((_IPW,), jnp.int32),
          pltpu.VMEM((_PER_W,), jnp.int32),
          pltpu.VMEM((2, _ROWS, _D), jnp.float32),
          pltpu.VMEM((2, _ROWS, _D), jnp.float32),
          pltpu.VMEM((2, _CH, _D), jnp.float32),
          pltpu.VMEM((_PER_W,), jnp.float32),
          pltpu.VMEM((_NEG, _PER_W), jnp.float32),
          pltpu.SemaphoreType.DMA,
          pltpu.SemaphoreType.DMA,
      ],
  )
  def scores(ctx_i_hbm, ctr_i_hbm, neg_i_hbm, ctx_t_hbm, ctr_t_hbm,
             pos_hbm, negs_hbm,
             ctxi_v, negi_v, ctri_v,
             ctx_rows_v, neg_rows_v, ctr_rows_v,
             pos_v, negs_v, sem0, sem1):
    wid = lax.axis_index("s") * _NC + lax.axis_index("c")
    lane = lax.iota(jnp.int32, 16)
    zero = jnp.zeros((16,), jnp.float32)

    pltpu.sync_copy(ctx_i_hbm.at[pl.ds(wid * _IPW, _IPW)], ctxi_v)
    pltpu.sync_copy(neg_i_hbm.at[pl.ds(wid * _IPW, _IPW)], negi_v)
    pltpu.sync_copy(ctr_i_hbm.at[pl.ds(wid * _PER_W, _PER_W)], ctri_v)

    def dma_descs(c, b, sem):
      ib = c * _ROWS
      ds = []
      for off, ln in _SPLITS:
        ds.append(pltpu.make_async_copy(
            ctx_t_hbm.at[ctxi_v.at[pl.ds(ib + off, ln)]],
            ctx_rows_v.at[b, pl.ds(off, ln)], sem))
        ds.append(pltpu.make_async_copy(
            ctr_t_hbm.at[negi_v.at[pl.ds(ib + off, ln)]],
            neg_rows_v.at[b, pl.ds(off, ln)], sem))
      ds.append(pltpu.make_async_copy(
          ctr_t_hbm.at[ctri_v.at[pl.ds(c * _CH, _CH)]],
          ctr_rows_v.at[b], sem))
      return ds

    def issue(c, b, sem):
      for d in dma_descs(c, b, sem):
        d.start()

    def drain(c, b, sem):
      for d in dma_descs(c, b, sem):
        d.wait()

    def compute(c, b):
      def elem(e, carry):
        pos_acc, neg_acc = carry
        base = e * _CTX
        macc = [zero] * _QS
        for r in range(_CTX):
          for q in range(_QS):
            macc[q] = macc[q] + ctx_rows_v[b, base + r, pl.ds(q * 16, 16)]
        dot = zero
        for q in range(_QS):
          dot = dot + macc[q] * ctr_rows_v[b, e, pl.ds(q * 16, 16)]
        mask = lane == e
        s = jnp.sum(dot) * jnp.float32(1.0 / _CTX)
        pos_acc = jnp.where(mask, jnp.full((16,), s, jnp.float32), pos_acc)
        new_neg = []
        for n in range(_NEG):
          dn = zero
          for q in range(_QS):
            dn = dn + macc[q] * neg_rows_v[b, base + n, pl.ds(q * 16, 16)]
          sn = jnp.sum(dn) * jnp.float32(1.0 / _CTX)
          new_neg.append(
              jnp.where(mask, jnp.full((16,), sn, jnp.float32), neg_acc[n]))
        return pos_acc, tuple(new_neg)

      pos_acc, neg_acc = lax.fori_loop(
          0, _CH, elem, (zero, tuple(zero for _ in range(_NEG))))
      off = c * _CH
      pos_v[pl.ds(off, 16)] = pos_acc
      for n in range(_NEG):
        negs_v[n, pl.ds(off, 16)] = neg_acc[n]

    issue(0, 0, sem0)

    def gbody(g, carry):
      issue(2 * g + 1, 1, sem1)
      drain(2 * g, 0, sem0)
      compute(2 * g, 0)

      @pl.when(g < _NCHUNK // 2 - 1)
      def _():
        issue(2 * g + 2, 0, sem0)

      drain(2 * g + 1, 1, sem1)
      compute(2 * g + 1, 1)
      return carry

    lax.fori_loop(0, _NCHUNK // 2, gbody, jnp.int32(0))
    pltpu.sync_copy(pos_v, pos_hbm.at[wid])
    pltpu.sync_copy(negs_v, negs_hbm.at[wid])

  return scores(ctx_idx, center_idx, neg_idx, ctx_t2, ctr_t2)


def _loss_tc(pos, negs):
  def body(pos_ref, neg_ref, out_ref):
    p = pos_ref[...]
    q = neg_ref[...]

    def ls(x):
      return jnp.minimum(x, 0.0) - jnp.log1p(jnp.exp(-jnp.abs(x)))

    total = jnp.sum(ls(p)) + jnp.sum(ls(-q))
    out_ref[...] = jnp.full((1, 1), -total / _B, jnp.float32)

  return pl.pallas_call(
      body,
      out_shape=jax.ShapeDtypeStruct((1, 1), jnp.float32),
  )(pos, negs)


def kernel(context, center, negatives, ctx_table, ctr_table):
  ctx_t2 = jnp.pad(ctx_table, ((0, 0), (0, 64))).reshape(2 * _V, _D)
  ctr_t2 = jnp.pad(ctr_table, ((0, 0), (0, 64))).reshape(2 * _V, _D)
  ctx_i = (context.astype(jnp.int32) * 2).reshape(_B * _CTX)
  neg_i = (negatives.astype(jnp.int32) * 2).reshape(_B * _NEG)
  ctr_i = center.astype(jnp.int32) * 2
  pos, negs = _sc_scores(ctx_i, ctr_i, neg_i, ctx_t2, ctr_t2)
  loss = _loss_tc(pos, negs.reshape(_NW * _NEG, _PER_W))
  return loss[0, 0]


# speedup vs baseline: 6.2832x; 1.0597x over previous
import functools

import jax
import jax.numpy as jnp
from jax import lax
from jax.experimental import pallas as pl
from jax.experimental.pallas import tpu as pltpu
from jax.experimental.pallas import tpu_sc as plsc

_V = 1000000
_B = 16384
_D = 64
_CTX = 20
_NEG = 20
_NC = 2
_NS = 16
_NW = _NC * _NS
_PER_W = _B // _NW
_CH = 16
_NCHUNK = _PER_W // _CH
_ROWS = _CH * _CTX
_IPW = _PER_W * _CTX
_QS = _D // 16
_SPLITS = ((0, 128), (128, 128), (256, 64))


def _sc_scores(ctx_idx, center_idx, neg_idx, ctx_t2, ctr_t2):
  mesh = plsc.VectorSubcoreMesh(core_axis_name="c", subcore_axis_name="s")

  @functools.partial(
      pl.kernel,
      out_type=(
          jax.ShapeDtypeStruct((_NW, _PER_W), jnp.float32),
          jax.ShapeDtypeStruct((_NW, _NEG, _PER_W), jnp.float32),
      ),
      mesh=mesh,
      compiler_params=pltpu.CompilerParams(
          needs_layout_passes=False, use_tc_tiling_on_sc=False),
      scratch_types=[
          pltpu.VMEM((_IPW,), jnp.int32),
          pltpu.VMEM((_IPW,), jnp.int32),
          pltpu.VMEM((_PER_W,), jnp.int32),
          pltpu.VMEM((2, _ROWS, _D), jnp.float32),
          pltpu.VMEM((2, _ROWS, _D), jnp.float32),
          pltpu.VMEM((2, _CH, _D), jnp.float32),
          pltpu.VMEM((_PER_W,), jnp.float32),
          pltpu.VMEM((_NEG, _PER_W), jnp.float32),
          pltpu.SemaphoreType.DMA,
          pltpu.SemaphoreType.DMA,
      ],
  )
  def scores(ctx_i_hbm, ctr_i_hbm, neg_i_hbm, ctx_t_hbm, ctr_t_hbm,
             pos_hbm, negs_hbm,
             ctxi_v, negi_v, ctri_v,
             ctx_rows_v, neg_rows_v, ctr_rows_v,
             pos_v, negs_v, sem0, sem1):
    wid = lax.axis_index("s") * _NC + lax.axis_index("c")
    lane = lax.iota(jnp.int32, 16)
    zero = jnp.zeros((16,), jnp.float32)

    pltpu.sync_copy(ctx_i_hbm.at[pl.ds(wid * _IPW, _IPW)], ctxi_v)
    pltpu.sync_copy(neg_i_hbm.at[pl.ds(wid * _IPW, _IPW)], negi_v)
    pltpu.sync_copy(ctr_i_hbm.at[pl.ds(wid * _PER_W, _PER_W)], ctri_v)

    def dma_descs(c, b, sem):
      ib = c * _ROWS
      ds = []
      for off, ln in _SPLITS:
        ds.append(pltpu.make_async_copy(
            ctx_t_hbm.at[ctxi_v.at[pl.ds(ib + off, ln)]],
            ctx_rows_v.at[b, pl.ds(off, ln)], sem))
        ds.append(pltpu.make_async_copy(
            ctr_t_hbm.at[negi_v.at[pl.ds(ib + off, ln)]],
            neg_rows_v.at[b, pl.ds(off, ln)], sem))
      ds.append(pltpu.make_async_copy(
          ctr_t_hbm.at[ctri_v.at[pl.ds(c * _CH, _CH)]],
          ctr_rows_v.at[b], sem))
      return ds

    def issue(c, b, sem):
      for d in dma_descs(c, b, sem):
        d.start()

    def drain(c, b, sem):
      for d in dma_descs(c, b, sem):
        d.wait()

    def compute(c, b):
      def elem(e, carry):
        pos_acc, neg_acc = carry
        base = e * _CTX
        macc = [zero] * _QS
        for r in range(_CTX):
          for q in range(_QS):
            macc[q] = macc[q] + ctx_rows_v[b, base + r, pl.ds(q * 16, 16)]
        dot = zero
        for q in range(_QS):
          dot = dot + macc[q] * ctr_rows_v[b, e, pl.ds(q * 16, 16)]
        mask = lane == e
        s = jnp.sum(dot) * jnp.float32(1.0 / _CTX)
        pos_acc = jnp.where(mask, jnp.full((16,), s, jnp.float32), pos_acc)
        new_neg = []
        for n in range(_NEG):
          dn = zero
          for q in range(_QS):
            dn = dn + macc[q] * neg_rows_v[b, base + n, pl.ds(q * 16, 16)]
          sn = jnp.sum(dn) * jnp.float32(1.0 / _CTX)
          new_neg.append(
              jnp.where(mask, jnp.full((16,), sn, jnp.float32), neg_acc[n]))
        return pos_acc, tuple(new_neg)

      pos_acc, neg_acc = lax.fori_loop(
          0, _CH, elem, (zero, tuple(zero for _ in range(_NEG))))
      off = c * _CH
      pos_v[pl.ds(off, 16)] = pos_acc
      for n in range(_NEG):
        negs_v[n, pl.ds(off, 16)] = neg_acc[n]

    issue(0, 0, sem0)

    def gbody(g, carry):
      issue(2 * g + 1, 1, sem1)
      drain(2 * g, 0, sem0)
      compute(2 * g, 0)

      @pl.when(g < _NCHUNK // 2 - 1)
      def _():
        issue(2 * g + 2, 0, sem0)

      drain(2 * g + 1, 1, sem1)
      compute(2 * g + 1, 1)
      return carry

    lax.fori_loop(0, _NCHUNK // 2, gbody, jnp.int32(0))
    pltpu.sync_copy(pos_v, pos_hbm.at[wid])
    pltpu.sync_copy(negs_v, negs_hbm.at[wid])

  return scores(ctx_idx, center_idx, neg_idx, ctx_t2, ctr_t2)


_VB = 2048


def _tc_convert(table):
  t_dv = table.T

  def body(in_ref, out_ref):
    out_ref[:, 0:_D] = in_ref[...].T

  out = pl.pallas_call(
      body,
      grid=(pl.cdiv(_V, _VB),),
      in_specs=[pl.BlockSpec((_D, _VB), lambda i: (0, i))],
      out_specs=pl.BlockSpec((_VB, 128), lambda i: (i, 0)),
      out_shape=jax.ShapeDtypeStruct((_V, 128), jnp.float32),
  )(t_dv)
  return out.reshape(2 * _V, _D)


def _loss_tc(pos, negs):
  def body(pos_ref, neg_ref, out_ref):
    p = pos_ref[...]
    q = neg_ref[...]

    def ls(x):
      return jnp.minimum(x, 0.0) - jnp.log1p(jnp.exp(-jnp.abs(x)))

    total = jnp.sum(ls(p)) + jnp.sum(ls(-q))
    out_ref[...] = jnp.full((1, 1), -total / _B, jnp.float32)

  return pl.pallas_call(
      body,
      out_shape=jax.ShapeDtypeStruct((1, 1), jnp.float32),
  )(pos, negs)


def kernel(context, center, negatives, ctx_table, ctr_table):
  ctx_t2 = _tc_convert(ctx_table)
  ctr_t2 = _tc_convert(ctr_table)
  ctx_i = (context.astype(jnp.int32) * 2).reshape(_B * _CTX)
  neg_i = (negatives.astype(jnp.int32) * 2).reshape(_B * _NEG)
  ctr_i = center.astype(jnp.int32) * 2
  pos, negs = _sc_scores(ctx_i, ctr_i, neg_i, ctx_t2, ctr_t2)
  loss = _loss_tc(pos, negs.reshape(_NW * _NEG, _PER_W))
  return loss[0, 0]


# speedup vs baseline: 8.0569x; 1.2823x over previous
import functools

import jax
import jax.numpy as jnp
from jax import lax
from jax.experimental import pallas as pl
from jax.experimental.pallas import tpu as pltpu
from jax.experimental.pallas import tpu_sc as plsc

_V = 1000000
_B = 16384
_D = 64
_CTX = 20
_NEG = 20
_NC = 2
_NS = 16
_NW = _NC * _NS
_PER_W = _B // _NW
_CH = 16
_NCHUNK = _PER_W // _CH
_ROWS = _CH * _CTX
_IPW = _PER_W * _CTX
_QS = _D // 16
_SPLITS = ((0, 128), (128, 128), (256, 64))


def _sc_scores(ctx_idx, center_idx, neg_idx, ctx_t2, ctr_t2):
  mesh = plsc.VectorSubcoreMesh(core_axis_name="c", subcore_axis_name="s")

  @functools.partial(
      pl.kernel,
      out_type=(
          jax.ShapeDtypeStruct((_NW, _PER_W), jnp.float32),
          jax.ShapeDtypeStruct((_NW, _NEG, _PER_W), jnp.float32),
      ),
      mesh=mesh,
      compiler_params=pltpu.CompilerParams(
          needs_layout_passes=False, use_tc_tiling_on_sc=False),
      scratch_types=[
          pltpu.VMEM((_IPW,), jnp.int32),
          pltpu.VMEM((_IPW,), jnp.int32),
          pltpu.VMEM((_PER_W,), jnp.int32),
          pltpu.VMEM((2, _ROWS, _D), jnp.float32),
          pltpu.VMEM((2, _ROWS, _D), jnp.float32),
          pltpu.VMEM((2, _CH, _D), jnp.float32),
          pltpu.VMEM((_PER_W,), jnp.float32),
          pltpu.VMEM((_NEG, _PER_W), jnp.float32),
          pltpu.SemaphoreType.DMA,
          pltpu.SemaphoreType.DMA,
      ],
  )
  def scores(ctx_i_hbm, ctr_i_hbm, neg_i_hbm, ctx_t_hbm, ctr_t_hbm,
             pos_hbm, negs_hbm,
             ctxi_v, negi_v, ctri_v,
             ctx_rows_v, neg_rows_v, ctr_rows_v,
             pos_v, negs_v, sem0, sem1):
    wid = lax.axis_index("s") * _NC + lax.axis_index("c")
    lane = lax.iota(jnp.int32, 16)
    zero = jnp.zeros((16,), jnp.float32)

    pltpu.sync_copy(ctx_i_hbm.at[pl.ds(wid * _IPW, _IPW)], ctxi_v)
    pltpu.sync_copy(neg_i_hbm.at[pl.ds(wid * _IPW, _IPW)], negi_v)
    pltpu.sync_copy(ctr_i_hbm.at[pl.ds(wid * _PER_W, _PER_W)], ctri_v)

    def dma_descs(c, b, sem):
      ib = c * _ROWS
      ds = []
      for off, ln in _SPLITS:
        ds.append(pltpu.make_async_copy(
            ctx_t_hbm.at[ctxi_v.at[pl.ds(ib + off, ln)]],
            ctx_rows_v.at[b, pl.ds(off, ln)], sem))
        ds.append(pltpu.make_async_copy(
            ctr_t_hbm.at[negi_v.at[pl.ds(ib + off, ln)]],
            neg_rows_v.at[b, pl.ds(off, ln)], sem))
      ds.append(pltpu.make_async_copy(
          ctr_t_hbm.at[ctri_v.at[pl.ds(c * _CH, _CH)]],
          ctr_rows_v.at[b], sem))
      return ds

    def issue(c, b, sem):
      for d in dma_descs(c, b, sem):
        d.start()

    def drain(c, b, sem):
      for d in dma_descs(c, b, sem):
        d.wait()

    def compute(c, b):
      def elem(e, carry):
        pos_acc, neg_acc = carry
        base = e * _CTX
        macc = [zero] * _QS
        for r in range(_CTX):
          for q in range(_QS):
            macc[q] = macc[q] + ctx_rows_v[b, base + r, pl.ds(q * 16, 16)]
        dot = zero
        for q in range(_QS):
          dot = dot + macc[q] * ctr_rows_v[b, e, pl.ds(q * 16, 16)]
        mask = lane == e
        s = jnp.sum(dot) * jnp.float32(1.0 / _CTX)
        pos_acc = jnp.where(mask, jnp.full((16,), s, jnp.float32), pos_acc)
        new_neg = []
        for n in range(_NEG):
          dn = zero
          for q in range(_QS):
            dn = dn + macc[q] * neg_rows_v[b, base + n, pl.ds(q * 16, 16)]
          sn = jnp.sum(dn) * jnp.float32(1.0 / _CTX)
          new_neg.append(
              jnp.where(mask, jnp.full((16,), sn, jnp.float32), neg_acc[n]))
        return pos_acc, tuple(new_neg)

      pos_acc, neg_acc = lax.fori_loop(
          0, _CH, elem, (zero, tuple(zero for _ in range(_NEG))))
      off = c * _CH
      pos_v[pl.ds(off, 16)] = pos_acc
      for n in range(_NEG):
        negs_v[n, pl.ds(off, 16)] = neg_acc[n]

    issue(0, 0, sem0)

    def gbody(g, carry):
      issue(2 * g + 1, 1, sem1)
      drain(2 * g, 0, sem0)
      compute(2 * g, 0)

      @pl.when(g < _NCHUNK // 2 - 1)
      def _():
        issue(2 * g + 2, 0, sem0)

      drain(2 * g + 1, 1, sem1)
      compute(2 * g + 1, 1)
      return carry

    lax.fori_loop(0, _NCHUNK // 2, gbody, jnp.int32(0))
    pltpu.sync_copy(pos_v, pos_hbm.at[wid])
    pltpu.sync_copy(negs_v, negs_hbm.at[wid])

  return scores(ctx_idx, center_idx, neg_idx, ctx_t2, ctr_t2)


_VB = 4096


def _tc_convert(table):
  t_dv = table.T

  def body(in_ref, out_ref):
    x = in_ref[...]
    eye = jnp.eye(_D, dtype=jnp.float32)
    out_ref[:, 0:_D] = jax.lax.dot_general(
        x, eye, dimension_numbers=(((0,), (0,)), ((), ())),
        preferred_element_type=jnp.float32)

  out = pl.pallas_call(
      body,
      grid=(pl.cdiv(_V, _VB),),
      in_specs=[pl.BlockSpec((_D, _VB), lambda i: (0, i))],
      out_specs=pl.BlockSpec((_VB, 128), lambda i: (i, 0)),
      out_shape=jax.ShapeDtypeStruct((_V, 128), jnp.float32),
  )(t_dv)
  return out.reshape(2 * _V, _D)


def _loss_tc(pos, negs):
  def body(pos_ref, neg_ref, out_ref):
    p = pos_ref[...]
    q = neg_ref[...]

    def ls(x):
      return jnp.minimum(x, 0.0) - jnp.log1p(jnp.exp(-jnp.abs(x)))

    total = jnp.sum(ls(p)) + jnp.sum(ls(-q))
    out_ref[...] = jnp.full((1, 1), -total / _B, jnp.float32)

  return pl.pallas_call(
      body,
      out_shape=jax.ShapeDtypeStruct((1, 1), jnp.float32),
  )(pos, negs)


def kernel(context, center, negatives, ctx_table, ctr_table):
  ctx_t2 = _tc_convert(ctx_table)
  ctr_t2 = _tc_convert(ctr_table)
  ctx_i = (context.astype(jnp.int32) * 2).reshape(_B * _CTX)
  neg_i = (negatives.astype(jnp.int32) * 2).reshape(_B * _NEG)
  ctr_i = center.astype(jnp.int32) * 2
  pos, negs = _sc_scores(ctx_i, ctr_i, neg_i, ctx_t2, ctr_t2)
  loss = _loss_tc(pos, negs.reshape(_NW * _NEG, _PER_W))
  return loss[0, 0]


# speedup vs baseline: 10.3944x; 1.2901x over previous
import functools

import jax
import jax.numpy as jnp
from jax import lax
from jax.experimental import pallas as pl
from jax.experimental.pallas import tpu as pltpu
from jax.experimental.pallas import tpu_sc as plsc

_V = 1000000
_B = 16384
_D = 64
_CTX = 20
_NEG = 20
_NC = 2
_NS = 16
_NW = _NC * _NS
_PER_W = _B // _NW
_CH = 16
_NCHUNK = _PER_W // _CH
_ROWS = _CH * _CTX
_IPW = _PER_W * _CTX
_QS = _D // 16
_SPLITS = ((0, 128), (128, 128), (256, 64))


def _sc_scores(ctx_idx, center_idx, neg_idx, tbl2):
  mesh = plsc.VectorSubcoreMesh(core_axis_name="c", subcore_axis_name="s")

  @functools.partial(
      pl.kernel,
      out_type=(
          jax.ShapeDtypeStruct((_NW, _PER_W), jnp.float32),
          jax.ShapeDtypeStruct((_NW, _NEG, _PER_W), jnp.float32),
      ),
      mesh=mesh,
      compiler_params=pltpu.CompilerParams(
          needs_layout_passes=False, use_tc_tiling_on_sc=False),
      scratch_types=[
          pltpu.VMEM((_IPW,), jnp.int32),
          pltpu.VMEM((_IPW,), jnp.int32),
          pltpu.VMEM((_PER_W,), jnp.int32),
          pltpu.VMEM((2, _ROWS, _D), jnp.float32),
          pltpu.VMEM((2, _ROWS, _D), jnp.float32),
          pltpu.VMEM((2, _CH, _D), jnp.float32),
          pltpu.VMEM((_PER_W,), jnp.float32),
          pltpu.VMEM((_NEG, _PER_W), jnp.float32),
          pltpu.SemaphoreType.DMA,
          pltpu.SemaphoreType.DMA,
      ],
  )
  def scores(ctx_i_hbm, ctr_i_hbm, neg_i_hbm, t_hbm,
             pos_hbm, negs_hbm,
             ctxi_v, negi_v, ctri_v,
             ctx_rows_v, neg_rows_v, ctr_rows_v,
             pos_v, negs_v, sem0, sem1):
    wid = lax.axis_index("s") * _NC + lax.axis_index("c")
    lane = lax.iota(jnp.int32, 16)
    zero = jnp.zeros((16,), jnp.float32)

    pltpu.sync_copy(ctx_i_hbm.at[pl.ds(wid * _IPW, _IPW)], ctxi_v)
    pltpu.sync_copy(neg_i_hbm.at[pl.ds(wid * _IPW, _IPW)], negi_v)
    pltpu.sync_copy(ctr_i_hbm.at[pl.ds(wid * _PER_W, _PER_W)], ctri_v)

    def dma_descs(c, b, sem):
      ib = c * _ROWS
      ds = []
      for off, ln in _SPLITS:
        ds.append(pltpu.make_async_copy(
            t_hbm.at[ctxi_v.at[pl.ds(ib + off, ln)]],
            ctx_rows_v.at[b, pl.ds(off, ln)], sem))
        ds.append(pltpu.make_async_copy(
            t_hbm.at[negi_v.at[pl.ds(ib + off, ln)]],
            neg_rows_v.at[b, pl.ds(off, ln)], sem))
      ds.append(pltpu.make_async_copy(
          t_hbm.at[ctri_v.at[pl.ds(c * _CH, _CH)]],
          ctr_rows_v.at[b], sem))
      return ds

    def issue(c, b, sem):
      for d in dma_descs(c, b, sem):
        d.start()

    def drain(c, b, sem):
      for d in dma_descs(c, b, sem):
        d.wait()

    def compute(c, b):
      def elem(e, carry):
        pos_acc, neg_acc = carry
        base = e * _CTX
        macc = [zero] * _QS
        for r in range(_CTX):
          for q in range(_QS):
            macc[q] = macc[q] + ctx_rows_v[b, base + r, pl.ds(q * 16, 16)]
        dot = zero
        for q in range(_QS):
          dot = dot + macc[q] * ctr_rows_v[b, e, pl.ds(q * 16, 16)]
        mask = lane == e
        s = jnp.sum(dot) * jnp.float32(1.0 / _CTX)
        pos_acc = jnp.where(mask, jnp.full((16,), s, jnp.float32), pos_acc)
        new_neg = []
        for n in range(_NEG):
          dn = zero
          for q in range(_QS):
            dn = dn + macc[q] * neg_rows_v[b, base + n, pl.ds(q * 16, 16)]
          sn = jnp.sum(dn) * jnp.float32(1.0 / _CTX)
          new_neg.append(
              jnp.where(mask, jnp.full((16,), sn, jnp.float32), neg_acc[n]))
        return pos_acc, tuple(new_neg)

      pos_acc, neg_acc = lax.fori_loop(
          0, _CH, elem, (zero, tuple(zero for _ in range(_NEG))))
      off = c * _CH
      pos_v[pl.ds(off, 16)] = pos_acc
      for n in range(_NEG):
        negs_v[n, pl.ds(off, 16)] = neg_acc[n]

    issue(0, 0, sem0)

    def gbody(g, carry):
      issue(2 * g + 1, 1, sem1)
      drain(2 * g, 0, sem0)
      compute(2 * g, 0)

      @pl.when(g < _NCHUNK // 2 - 1)
      def _():
        issue(2 * g + 2, 0, sem0)

      drain(2 * g + 1, 1, sem1)
      compute(2 * g + 1, 1)
      return carry

    lax.fori_loop(0, _NCHUNK // 2, gbody, jnp.int32(0))
    pltpu.sync_copy(pos_v, pos_hbm.at[wid])
    pltpu.sync_copy(negs_v, negs_hbm.at[wid])

  return scores(ctx_idx, center_idx, neg_idx, tbl2)


_VB = 4096


def _tc_convert(ctx_table, ctr_table):
  a = ctx_table.T
  b = ctr_table.T

  def body(a_ref, b_ref, out_ref):
    eye = jnp.eye(_D, dtype=jnp.float32)
    dn = (((0,), (0,)), ((), ()))
    ya = jax.lax.dot_general(a_ref[...], eye, dimension_numbers=dn,
                             preferred_element_type=jnp.float32)
    yb = jax.lax.dot_general(b_ref[...], eye, dimension_numbers=dn,
                             preferred_element_type=jnp.float32)
    out_ref[...] = jnp.concatenate([ya, yb], axis=1)

  out = pl.pallas_call(
      body,
      grid=(pl.cdiv(_V, _VB),),
      in_specs=[pl.BlockSpec((_D, _VB), lambda i: (0, i)),
                pl.BlockSpec((_D, _VB), lambda i: (0, i))],
      out_specs=pl.BlockSpec((_VB, 128), lambda i: (i, 0)),
      out_shape=jax.ShapeDtypeStruct((_V, 128), jnp.float32),
  )(a, b)
  return out.reshape(2 * _V, _D)


def _loss_tc(pos, negs):
  def body(pos_ref, neg_ref, out_ref):
    p = pos_ref[...]
    q = neg_ref[...]

    def ls(x):
      return jnp.minimum(x, 0.0) - jnp.log1p(jnp.exp(-jnp.abs(x)))

    total = jnp.sum(ls(p)) + jnp.sum(ls(-q))
    out_ref[...] = jnp.full((1, 1), -total / _B, jnp.float32)

  return pl.pallas_call(
      body,
      out_shape=jax.ShapeDtypeStruct((1, 1), jnp.float32),
  )(pos, negs)


def kernel(context, center, negatives, ctx_table, ctr_table):
  tbl2 = _tc_convert(ctx_table, ctr_table)
  ctx_i = (context.astype(jnp.int32) * 2).reshape(_B * _CTX)
  neg_i = (negatives.astype(jnp.int32) * 2 + 1).reshape(_B * _NEG)
  ctr_i = center.astype(jnp.int32) * 2 + 1
  pos, negs = _sc_scores(ctx_i, ctr_i, neg_i, tbl2)
  loss = _loss_tc(pos, negs.reshape(_NW * _NEG, _PER_W))
  return loss[0, 0]


# speedup vs baseline: 13.0057x; 1.2512x over previous
import functools

import jax
import jax.numpy as jnp
from jax import lax
from jax.experimental import pallas as pl
from jax.experimental.pallas import tpu as pltpu
from jax.experimental.pallas import tpu_sc as plsc

_V = 1000000
_B = 16384
_D = 64
_CTX = 20
_NEG = 20
_NC = 2
_NS = 16
_NW = _NC * _NS
_PER_W = _B // _NW
_CH = 16
_NCHUNK = _PER_W // _CH
_ROWS = _CH * _CTX
_IPW = _PER_W * _CTX
_QS = _D // 16
_SPLITS = ((0, 128), (128, 128), (256, 64))


def _sc_scores(ctx_idx, center_idx, neg_idx, tbl2):
  mesh = plsc.VectorSubcoreMesh(core_axis_name="c", subcore_axis_name="s")

  @functools.partial(
      pl.kernel,
      out_type=(
          jax.ShapeDtypeStruct((_NW, _PER_W), jnp.float32),
          jax.ShapeDtypeStruct((_NW, _NEG, _PER_W), jnp.float32),
      ),
      mesh=mesh,
      compiler_params=pltpu.CompilerParams(
          needs_layout_passes=False, use_tc_tiling_on_sc=False),
      scratch_types=[
          pltpu.VMEM((_IPW,), jnp.int32),
          pltpu.VMEM((_IPW,), jnp.int32),
          pltpu.VMEM((_PER_W,), jnp.int32),
          pltpu.VMEM((2, _ROWS, _D), jnp.float32),
          pltpu.VMEM((2, _ROWS, _D), jnp.float32),
          pltpu.VMEM((2, _CH, _D), jnp.float32),
          pltpu.VMEM((_PER_W,), jnp.float32),
          pltpu.VMEM((_NEG, _PER_W), jnp.float32),
          pltpu.SemaphoreType.DMA,
          pltpu.SemaphoreType.DMA,
      ],
  )
  def scores(ctx_i_hbm, ctr_i_hbm, neg_i_hbm, t_hbm,
             pos_hbm, negs_hbm,
             ctxi_v, negi_v, ctri_v,
             ctx_rows_v, neg_rows_v, ctr_rows_v,
             pos_v, negs_v, sem0, sem1):
    wid = lax.axis_index("s") * _NC + lax.axis_index("c")
    lane = lax.iota(jnp.int32, 16)
    zero = jnp.zeros((16,), jnp.float32)

    pltpu.sync_copy(ctx_i_hbm.at[pl.ds(wid * _IPW, _IPW)], ctxi_v)
    pltpu.sync_copy(neg_i_hbm.at[pl.ds(wid * _IPW, _IPW)], negi_v)
    pltpu.sync_copy(ctr_i_hbm.at[pl.ds(wid * _PER_W, _PER_W)], ctri_v)

    def dma_descs(c, b, sem):
      ib = c * _ROWS
      ds = []
      for off, ln in _SPLITS:
        ds.append(pltpu.make_async_copy(
            t_hbm.at[ctxi_v.at[pl.ds(ib + off, ln)]],
            ctx_rows_v.at[b, pl.ds(off, ln)], sem))
        ds.append(pltpu.make_async_copy(
            t_hbm.at[negi_v.at[pl.ds(ib + off, ln)]],
            neg_rows_v.at[b, pl.ds(off, ln)], sem))
      ds.append(pltpu.make_async_copy(
          t_hbm.at[ctri_v.at[pl.ds(c * _CH, _CH)]],
          ctr_rows_v.at[b], sem))
      return ds

    def issue(c, b, sem):
      for d in dma_descs(c, b, sem):
        d.start()

    def drain(c, b, sem):
      for d in dma_descs(c, b, sem):
        d.wait()

    def compute(c, b):
      def elem(e, carry):
        pos_acc, neg_acc = carry
        base = e * _CTX
        macc = [zero] * _QS
        for r in range(_CTX):
          for q in range(_QS):
            macc[q] = macc[q] + ctx_rows_v[b, base + r, pl.ds(q * 16, 16)]
        dot = zero
        for q in range(_QS):
          dot = dot + macc[q] * ctr_rows_v[b, e, pl.ds(q * 16, 16)]
        mask = lane == e
        s = jnp.sum(dot) * jnp.float32(1.0 / _CTX)
        pos_acc = jnp.where(mask, jnp.full((16,), s, jnp.float32), pos_acc)
        new_neg = []
        for n in range(_NEG):
          dn = zero
          for q in range(_QS):
            dn = dn + macc[q] * neg_rows_v[b, base + n, pl.ds(q * 16, 16)]
          sn = jnp.sum(dn) * jnp.float32(1.0 / _CTX)
          new_neg.append(
              jnp.where(mask, jnp.full((16,), sn, jnp.float32), neg_acc[n]))
        return pos_acc, tuple(new_neg)

      pos_acc, neg_acc = lax.fori_loop(
          0, _CH, elem, (zero, tuple(zero for _ in range(_NEG))))
      off = c * _CH
      pos_v[pl.ds(off, 16)] = pos_acc
      for n in range(_NEG):
        negs_v[n, pl.ds(off, 16)] = neg_acc[n]

    issue(0, 0, sem0)

    def gbody(g, carry):
      issue(2 * g + 1, 1, sem1)
      drain(2 * g, 0, sem0)
      compute(2 * g, 0)

      @pl.when(g < _NCHUNK // 2 - 1)
      def _():
        issue(2 * g + 2, 0, sem0)

      drain(2 * g + 1, 1, sem1)
      compute(2 * g + 1, 1)
      return carry

    lax.fori_loop(0, _NCHUNK // 2, gbody, jnp.int32(0))
    pltpu.sync_copy(pos_v, pos_hbm.at[wid])
    pltpu.sync_copy(negs_v, negs_hbm.at[wid])

  return scores(ctx_idx, center_idx, neg_idx, tbl2)


_VB = 4096


def _tc_convert(ctx_table, ctr_table):
  a = ctx_table.T
  b = ctr_table.T

  def body(a_ref, b_ref, out_ref):
    x = jnp.concatenate([a_ref[...], b_ref[...]], axis=0)
    eye = jnp.eye(2 * _D, dtype=jnp.float32)
    out_ref[...] = jax.lax.dot_general(
        x, eye, dimension_numbers=(((0,), (0,)), ((), ())),
        preferred_element_type=jnp.float32)

  out = pl.pallas_call(
      body,
      grid=(pl.cdiv(_V, _VB),),
      in_specs=[pl.BlockSpec((_D, _VB), lambda i: (0, i)),
                pl.BlockSpec((_D, _VB), lambda i: (0, i))],
      out_specs=pl.BlockSpec((_VB, 128), lambda i: (i, 0)),
      out_shape=jax.ShapeDtypeStruct((_V, 128), jnp.float32),
  )(a, b)
  return out.reshape(2 * _V, _D)


def _loss_tc(pos, negs):
  def body(pos_ref, neg_ref, out_ref):
    p = pos_ref[...]
    q = neg_ref[...]

    def ls(x):
      return jnp.minimum(x, 0.0) - jnp.log1p(jnp.exp(-jnp.abs(x)))

    total = jnp.sum(ls(p)) + jnp.sum(ls(-q))
    out_ref[...] = jnp.full((1, 1), -total / _B, jnp.float32)

  return pl.pallas_call(
      body,
      out_shape=jax.ShapeDtypeStruct((1, 1), jnp.float32),
  )(pos, negs)


def kernel(context, center, negatives, ctx_table, ctr_table):
  tbl2 = _tc_convert(ctx_table, ctr_table)
  ctx_i = (context.astype(jnp.int32) * 2).reshape(_B * _CTX)
  neg_i = (negatives.astype(jnp.int32) * 2 + 1).reshape(_B * _NEG)
  ctr_i = center.astype(jnp.int32) * 2 + 1
  pos, negs = _sc_scores(ctx_i, ctr_i, neg_i, tbl2)
  loss = _loss_tc(pos, negs.reshape(_NW * _NEG, _PER_W))
  return loss[0, 0]


# speedup vs baseline: 14.6108x; 1.1234x over previous
import functools

import jax
import jax.numpy as jnp
from jax import lax
from jax.experimental import pallas as pl
from jax.experimental.pallas import tpu as pltpu
from jax.experimental.pallas import tpu_sc as plsc

_V = 1000000
_B = 16384
_D = 64
_CTX = 20
_NEG = 20
_NC = 2
_NS = 16
_NW = _NC * _NS
_PER_W = _B // _NW
_CH = 16
_NCHUNK = _PER_W // _CH
_ROWS = _CH * _CTX
_IPW = _PER_W * _CTX
_QS = _D // 16
_SPLITS = ((0, 128), (128, 128), (256, 64))


def _sc_scores(ctx_idx, center_idx, neg_idx, tbl2):
  mesh = plsc.VectorSubcoreMesh(core_axis_name="c", subcore_axis_name="s")

  @functools.partial(
      pl.kernel,
      out_type=(
          jax.ShapeDtypeStruct((_NW, _PER_W), jnp.float32),
          jax.ShapeDtypeStruct((_NW, _NEG, _PER_W), jnp.float32),
      ),
      mesh=mesh,
      compiler_params=pltpu.CompilerParams(
          needs_layout_passes=False, use_tc_tiling_on_sc=False),
      scratch_types=[
          pltpu.VMEM((_IPW,), jnp.int32),
          pltpu.VMEM((_IPW,), jnp.int32),
          pltpu.VMEM((_PER_W,), jnp.int32),
          pltpu.VMEM((2, _ROWS, _D), jnp.float32),
          pltpu.VMEM((2, _ROWS, _D), jnp.float32),
          pltpu.VMEM((2, _CH, _D), jnp.float32),
          pltpu.VMEM((_PER_W,), jnp.float32),
          pltpu.VMEM((_NEG, _PER_W), jnp.float32),
          pltpu.SemaphoreType.DMA,
          pltpu.SemaphoreType.DMA,
      ],
  )
  def scores(ctx_i_hbm, ctr_i_hbm, neg_i_hbm, t_hbm,
             pos_hbm, negs_hbm,
             ctxi_v, negi_v, ctri_v,
             ctx_rows_v, neg_rows_v, ctr_rows_v,
             pos_v, negs_v, sem0, sem1):
    wid = lax.axis_index("s") * _NC + lax.axis_index("c")
    lane = lax.iota(jnp.int32, 16)
    zero = jnp.zeros((16,), jnp.float32)

    pltpu.sync_copy(ctx_i_hbm.at[pl.ds(wid * _IPW, _IPW)], ctxi_v)
    pltpu.sync_copy(neg_i_hbm.at[pl.ds(wid * _IPW, _IPW)], negi_v)
    pltpu.sync_copy(ctr_i_hbm.at[pl.ds(wid * _PER_W, _PER_W)], ctri_v)

    def dma_descs(c, b, sem):
      ib = c * _ROWS
      ds = []
      for off, ln in _SPLITS:
        ds.append(pltpu.make_async_copy(
            t_hbm.at[ctxi_v.at[pl.ds(ib + off, ln)]],
            ctx_rows_v.at[b, pl.ds(off, ln)], sem))
        ds.append(pltpu.make_async_copy(
            t_hbm.at[negi_v.at[pl.ds(ib + off, ln)]],
            neg_rows_v.at[b, pl.ds(off, ln)], sem))
      ds.append(pltpu.make_async_copy(
          t_hbm.at[ctri_v.at[pl.ds(c * _CH, _CH)]],
          ctr_rows_v.at[b], sem))
      return ds

    def issue(c, b, sem):
      for d in dma_descs(c, b, sem):
        d.start()

    def drain(c, b, sem):
      for d in dma_descs(c, b, sem):
        d.wait()

    def compute(c, b):
      def elem(e, carry):
        pos_acc, neg_acc = carry
        base = e * _CTX
        macc = [zero] * _QS
        for r in range(_CTX):
          for q in range(_QS):
            macc[q] = macc[q] + ctx_rows_v[b, base + r, pl.ds(q * 16, 16)]
        dot = zero
        for q in range(_QS):
          dot = dot + macc[q] * ctr_rows_v[b, e, pl.ds(q * 16, 16)]
        mask = lane == e
        s = jnp.sum(dot) * jnp.float32(1.0 / _CTX)
        pos_acc = jnp.where(mask, jnp.full((16,), s, jnp.float32), pos_acc)
        new_neg = []
        for n in range(_NEG):
          dn = zero
          for q in range(_QS):
            dn = dn + macc[q] * neg_rows_v[b, base + n, pl.ds(q * 16, 16)]
          sn = jnp.sum(dn) * jnp.float32(1.0 / _CTX)
          new_neg.append(
              jnp.where(mask, jnp.full((16,), sn, jnp.float32), neg_acc[n]))
        return pos_acc, tuple(new_neg)

      pos_acc, neg_acc = lax.fori_loop(
          0, _CH, elem, (zero, tuple(zero for _ in range(_NEG))))
      off = c * _CH
      pos_v[pl.ds(off, 16)] = pos_acc
      for n in range(_NEG):
        negs_v[n, pl.ds(off, 16)] = neg_acc[n]

    issue(0, 0, sem0)

    def gbody(g, carry):
      issue(2 * g + 1, 1, sem1)
      drain(2 * g, 0, sem0)
      compute(2 * g, 0)

      @pl.when(g < _NCHUNK // 2 - 1)
      def _():
        issue(2 * g + 2, 0, sem0)

      drain(2 * g + 1, 1, sem1)
      compute(2 * g + 1, 1)
      return carry

    lax.fori_loop(0, _NCHUNK // 2, gbody, jnp.int32(0))
    pltpu.sync_copy(pos_v, pos_hbm.at[wid])
    pltpu.sync_copy(negs_v, negs_hbm.at[wid])

  return scores(ctx_idx, center_idx, neg_idx, tbl2)


_VB = 8192


def _tc_convert(ctx_table, ctr_table):
  a = ctx_table.T
  b = ctr_table.T

  def body(a_ref, b_ref, out_ref):
    x = jnp.concatenate([a_ref[...], b_ref[...]], axis=0)
    eye = jnp.eye(2 * _D, dtype=jnp.float32)
    out_ref[...] = jax.lax.dot_general(
        x, eye, dimension_numbers=(((0,), (0,)), ((), ())),
        preferred_element_type=jnp.float32)

  out = pl.pallas_call(
      body,
      grid=(pl.cdiv(_V, _VB),),
      in_specs=[pl.BlockSpec((_D, _VB), lambda i: (0, i)),
                pl.BlockSpec((_D, _VB), lambda i: (0, i))],
      out_specs=pl.BlockSpec((_VB, 128), lambda i: (i, 0)),
      out_shape=jax.ShapeDtypeStruct((_V, 128), jnp.float32),
  )(a, b)
  return out.reshape(2 * _V, _D)


def _loss_tc(pos, negs):
  def body(pos_ref, neg_ref, out_ref):
    p = pos_ref[...]
    q = neg_ref[...]

    def ls(x):
      return jnp.minimum(x, 0.0) - jnp.log1p(jnp.exp(-jnp.abs(x)))

    total = jnp.sum(ls(p)) + jnp.sum(ls(-q))
    out_ref[...] = jnp.full((1, 1), -total / _B, jnp.float32)

  return pl.pallas_call(
      body,
      out_shape=jax.ShapeDtypeStruct((1, 1), jnp.float32),
  )(pos, negs)


def kernel(context, center, negatives, ctx_table, ctr_table):
  tbl2 = _tc_convert(ctx_table, ctr_table)
  ctx_i = (context.astype(jnp.int32) * 2).reshape(_B * _CTX)
  neg_i = (negatives.astype(jnp.int32) * 2 + 1).reshape(_B * _NEG)
  ctr_i = center.astype(jnp.int32) * 2 + 1
  pos, negs = _sc_scores(ctx_i, ctr_i, neg_i, tbl2)
  loss = _loss_tc(pos, negs.reshape(_NW * _NEG, _PER_W))
  return loss[0, 0]


# speedup vs baseline: 14.8628x; 1.0173x over previous
import functools

import jax
import jax.numpy as jnp
from jax import lax
from jax.experimental import pallas as pl
from jax.experimental.pallas import tpu as pltpu
from jax.experimental.pallas import tpu_sc as plsc

_V = 1000000
_B = 16384
_D = 64
_CTX = 20
_NEG = 20
_NC = 2
_NS = 16
_NW = _NC * _NS
_PER_W = _B // _NW
_CH = 16
_NCHUNK = _PER_W // _CH
_ROWS = _CH * _CTX
_IPW = _PER_W * _CTX
_QS = _D // 16
_SPLITS = ((0, 128), (128, 128), (256, 64))


def _sc_scores(ctx_idx, center_idx, neg_idx, tbl2):
  mesh = plsc.VectorSubcoreMesh(core_axis_name="c", subcore_axis_name="s")

  @functools.partial(
      pl.kernel,
      out_type=(
          jax.ShapeDtypeStruct((_NW, _PER_W), jnp.float32),
          jax.ShapeDtypeStruct((_NW, _NEG, _PER_W), jnp.float32),
      ),
      mesh=mesh,
      compiler_params=pltpu.CompilerParams(
          needs_layout_passes=False, use_tc_tiling_on_sc=False),
      scratch_types=[
          pltpu.VMEM((_IPW,), jnp.int32),
          pltpu.VMEM((_IPW,), jnp.int32),
          pltpu.VMEM((_PER_W,), jnp.int32),
          pltpu.VMEM((2, _ROWS, _D), jnp.float32),
          pltpu.VMEM((2, _ROWS, _D), jnp.float32),
          pltpu.VMEM((2, _CH, _D), jnp.float32),
          pltpu.VMEM((_PER_W,), jnp.float32),
          pltpu.VMEM((_NEG, _PER_W), jnp.float32),
          pltpu.SemaphoreType.DMA,
          pltpu.SemaphoreType.DMA,
      ],
  )
  def scores(ctx_i_hbm, ctr_i_hbm, neg_i_hbm, t_hbm,
             pos_hbm, negs_hbm,
             ctxi_v, negi_v, ctri_v,
             ctx_rows_v, neg_rows_v, ctr_rows_v,
             pos_v, negs_v, sem0, sem1):
    wid = lax.axis_index("s") * _NC + lax.axis_index("c")
    lane = lax.iota(jnp.int32, 16)
    zero = jnp.zeros((16,), jnp.float32)

    pltpu.sync_copy(ctx_i_hbm.at[pl.ds(wid * _IPW, _IPW)], ctxi_v)
    pltpu.sync_copy(neg_i_hbm.at[pl.ds(wid * _IPW, _IPW)], negi_v)
    pltpu.sync_copy(ctr_i_hbm.at[pl.ds(wid * _PER_W, _PER_W)], ctri_v)

    def dma_descs(c, b, sem):
      ib = c * _ROWS
      ds = []
      for off, ln in _SPLITS:
        ds.append(pltpu.make_async_copy(
            t_hbm.at[ctxi_v.at[pl.ds(ib + off, ln)]],
            ctx_rows_v.at[b, pl.ds(off, ln)], sem))
        ds.append(pltpu.make_async_copy(
            t_hbm.at[negi_v.at[pl.ds(ib + off, ln)]],
            neg_rows_v.at[b, pl.ds(off, ln)], sem))
      ds.append(pltpu.make_async_copy(
          t_hbm.at[ctri_v.at[pl.ds(c * _CH, _CH)]],
          ctr_rows_v.at[b], sem))
      return ds

    def issue(c, b, sem):
      for d in dma_descs(c, b, sem):
        d.start()

    def drain(c, b, sem):
      for d in dma_descs(c, b, sem):
        d.wait()

    def compute(c, b):
      def elem(e, carry):
        pos_acc, neg_acc = carry
        base = e * _CTX
        macc = [zero] * _QS
        for r in range(_CTX):
          for q in range(_QS):
            macc[q] = macc[q] + ctx_rows_v[b, base + r, pl.ds(q * 16, 16)]
        dot = zero
        for q in range(_QS):
          dot = dot + macc[q] * ctr_rows_v[b, e, pl.ds(q * 16, 16)]
        mask = lane == e
        s = jnp.sum(dot) * jnp.float32(1.0 / _CTX)
        pos_acc = jnp.where(mask, jnp.full((16,), s, jnp.float32), pos_acc)
        new_neg = []
        for n in range(_NEG):
          dn = zero
          for q in range(_QS):
            dn = dn + macc[q] * neg_rows_v[b, base + n, pl.ds(q * 16, 16)]
          sn = jnp.sum(dn) * jnp.float32(1.0 / _CTX)
          new_neg.append(
              jnp.where(mask, jnp.full((16,), sn, jnp.float32), neg_acc[n]))
        return pos_acc, tuple(new_neg)

      pos_acc, neg_acc = lax.fori_loop(
          0, _CH, elem, (zero, tuple(zero for _ in range(_NEG))))
      off = c * _CH
      pos_v[pl.ds(off, 16)] = pos_acc
      for n in range(_NEG):
        negs_v[n, pl.ds(off, 16)] = neg_acc[n]

    issue(0, 0, sem0)

    def gbody(g, carry):
      issue(2 * g + 1, 1, sem1)
      drain(2 * g, 0, sem0)
      compute(2 * g, 0)

      @pl.when(g < _NCHUNK // 2 - 1)
      def _():
        issue(2 * g + 2, 0, sem0)

      drain(2 * g + 1, 1, sem1)
      compute(2 * g + 1, 1)
      return carry

    lax.fori_loop(0, _NCHUNK // 2, gbody, jnp.int32(0))
    pltpu.sync_copy(pos_v, pos_hbm.at[wid])
    pltpu.sync_copy(negs_v, negs_hbm.at[wid])

  return scores(ctx_idx, center_idx, neg_idx, tbl2)


_VB = 16384


def _tc_convert(ctx_table, ctr_table):
  a = ctx_table.T
  b = ctr_table.T

  def body(a_ref, b_ref, out_ref):
    x = jnp.concatenate([a_ref[...], b_ref[...]], axis=0)
    eye = jnp.eye(2 * _D, dtype=jnp.float32)
    out_ref[...] = jax.lax.dot_general(
        x, eye, dimension_numbers=(((0,), (0,)), ((), ())),
        preferred_element_type=jnp.float32)

  out = pl.pallas_call(
      body,
      grid=(pl.cdiv(_V, _VB),),
      in_specs=[pl.BlockSpec((_D, _VB), lambda i: (0, i)),
                pl.BlockSpec((_D, _VB), lambda i: (0, i))],
      out_specs=pl.BlockSpec((_VB, 128), lambda i: (i, 0)),
      out_shape=jax.ShapeDtypeStruct((_V, 128), jnp.float32),
  )(a, b)
  return out.reshape(2 * _V, _D)


def _loss_tc(pos, negs):
  def body(pos_ref, neg_ref, out_ref):
    p = pos_ref[...]
    q = neg_ref[...]

    def ls(x):
      return jnp.minimum(x, 0.0) - jnp.log1p(jnp.exp(-jnp.abs(x)))

    total = jnp.sum(ls(p)) + jnp.sum(ls(-q))
    out_ref[...] = jnp.full((1, 1), -total / _B, jnp.float32)

  return pl.pallas_call(
      body,
      out_shape=jax.ShapeDtypeStruct((1, 1), jnp.float32),
  )(pos, negs)


def kernel(context, center, negatives, ctx_table, ctr_table):
  tbl2 = _tc_convert(ctx_table, ctr_table)
  ctx_i = (context.astype(jnp.int32) * 2).reshape(_B * _CTX)
  neg_i = (negatives.astype(jnp.int32) * 2 + 1).reshape(_B * _NEG)
  ctr_i = center.astype(jnp.int32) * 2 + 1
  pos, negs = _sc_scores(ctx_i, ctr_i, neg_i, tbl2)
  loss = _loss_tc(pos, negs.reshape(_NW * _NEG, _PER_W))
  return loss[0, 0]
